# baseline (speedup 1.0000x reference)
import functools

import jax
import jax.numpy as jnp
from jax import lax
from jax.experimental import pallas as pl
from jax.experimental.pallas import tpu as pltpu
from jax.experimental.pallas import tpu_sc as plsc

N = 10000
E = 320000
D = 128
NCORES = 2
NSUB = 16
NTILES = NCORES * NSUB
EPT = E // NTILES
B = 80
NCH = EPT // B
NG = B // 16


def _tc_front(x, W_v, aq2, ak2):
    def body(x_ref, w_ref, aq_ref, ak_ref, xv_ref, q_ref, k_ref):
        xv = jnp.dot(x_ref[...], w_ref[...], preferred_element_type=jnp.float32)
        xv_ref[...] = xv
        q_ref[...] = jnp.dot(xv, aq_ref[...], preferred_element_type=jnp.float32)
        k_ref[...] = jnp.dot(xv, ak_ref[...], preferred_element_type=jnp.float32)

    return pl.pallas_call(
        body,
        out_shape=(
            jax.ShapeDtypeStruct((N, D), jnp.float32),
            jax.ShapeDtypeStruct((N, 1), jnp.float32),
            jax.ShapeDtypeStruct((N, 1), jnp.float32),
        ),
    )(x, W_v, aq2, ak2)


def _sc_logits(row, col, q, k):
    mesh = plsc.VectorSubcoreMesh(
        core_axis_name="c", subcore_axis_name="s", num_cores=NCORES
    )

    @functools.partial(
        pl.kernel,
        out_type=(
            jax.ShapeDtypeStruct((E,), jnp.float32),
            jax.ShapeDtypeStruct((NTILES, N), jnp.float32),
        ),
        mesh=mesh,
        compiler_params=pltpu.CompilerParams(needs_layout_passes=False),
        scratch_types=[
            pltpu.VMEM((N,), jnp.float32),
            pltpu.VMEM((N,), jnp.float32),
            pltpu.VMEM((N,), jnp.float32),
            pltpu.VMEM((B,), jnp.int32),
            pltpu.VMEM((B,), jnp.int32),
            pltpu.VMEM((B,), jnp.float32),
            pltpu.VMEM((B,), jnp.int32),
            pltpu.VMEM((B,), jnp.int32),
            pltpu.VMEM((B,), jnp.float32),
            pltpu.SemaphoreType.DMA,
            pltpu.SemaphoreType.DMA,
            pltpu.SemaphoreType.DMA,
            pltpu.SemaphoreType.DMA,
            pltpu.SemaphoreType.DMA,
            pltpu.SemaphoreType.DMA,
        ],
    )
    def sck(row_hbm, col_hbm, q_hbm, k_hbm, ex_hbm, s_out,
            q_loc, k_loc, s_loc,
            idxr0, idxc0, exb0, idxr1, idxc1, exb1,
            ir0, ic0, we0, ir1, ic1, we1):
        cid = lax.axis_index("c")
        sid = lax.axis_index("s")
        wid = cid * NSUB + sid

        pltpu.sync_copy(q_hbm, q_loc)
        pltpu.sync_copy(k_hbm, k_loc)

        zero16 = jnp.zeros((16,), jnp.float32)

        def zinit(i, c0):
            s_loc[pl.ds(i * 16, 16)] = zero16
            return c0

        lax.fori_loop(0, N // 16, zinit, 0)

        base = wid * EPT
        lane = lax.iota(jnp.int32, 16)
        bufs = (
            (idxr0, idxc0, exb0, ir0, ic0, we0),
            (idxr1, idxc1, exb1, ir1, ic1, we1),
        )

        def issue_idx(s, ci):
            idxr, idxc, _, ir, ic, _ = bufs[s]
            off = base + ci * B
            pltpu.async_copy(row_hbm.at[pl.ds(off, B)], idxr, ir)
            pltpu.async_copy(col_hbm.at[pl.ds(off, B)], idxc, ic)

        def wait_idx(s):
            idxr, idxc, _, ir, ic, _ = bufs[s]
            pltpu.make_async_copy(row_hbm.at[pl.ds(0, B)], idxr, ir).wait()
            pltpu.make_async_copy(col_hbm.at[pl.ds(0, B)], idxc, ic).wait()

        def issue_exw(s, ci):
            exb, we = bufs[s][2], bufs[s][5]
            pltpu.async_copy(exb, ex_hbm.at[pl.ds(base + ci * B, B)], we)

        def wait_exw(s):
            exb, we = bufs[s][2], bufs[s][5]
            pltpu.make_async_copy(exb, ex_hbm.at[pl.ds(0, B)], we).wait()

        def compute(s):
            idxr, idxc, exb = bufs[s][0], bufs[s][1], bufs[s][2]
            for g in range(NG):
                r16 = idxr[pl.ds(g * 16, 16)]
                c16 = idxc[pl.ds(g * 16, 16)]
                qv = plsc.load_gather(q_loc, [r16])
                kv = plsc.load_gather(k_loc, [c16])
                e = qv + kv
                e = jnp.maximum(e, 0.2 * e)
                ex16 = jnp.exp(e)
                exb[pl.ds(g * 16, 16)] = ex16
                for l in range(16):
                    plsc.addupdate_scatter(
                        s_loc, [r16], ex16, mask=lane == l
                    )

        def step(i, k, first, last):
            if first:
                @pl.when(i >= 2)
                def _():
                    wait_exw(k)
            else:
                wait_exw(k)
            compute(k)
            issue_exw(k, i)
            wait_idx(1 - k)
            if last:
                @pl.when(i + 2 < NCH)
                def _():
                    issue_idx(k, i + 2)
            else:
                issue_idx(k, i + 2)

        issue_idx(0, 0)
        wait_idx(0)
        issue_idx(1, 1)

        def pair(t, carry):
            i = 2 * t
            step(i, 0, first=True, last=False)
            step(i + 1, 1, first=True, last=True)
            return carry

        lax.fori_loop(0, NCH // 2, pair, 0)
        wait_exw(0)
        compute(0)
        issue_exw(0, NCH - 1)
        wait_exw(1)
        wait_exw(0)

        pltpu.sync_copy(s_loc, s_out.at[wid])

    return sck(row, col, q, k)


def _sc_spmm(row, col, ex, xv, zeros_init):
    mesh = plsc.VectorSubcoreMesh(
        core_axis_name="c", subcore_axis_name="s", num_cores=NCORES
    )

    slot_vmem = [
        pltpu.VMEM((B,), jnp.int32),
        pltpu.VMEM((B,), jnp.int32),
        pltpu.VMEM((B,), jnp.int32),
        pltpu.VMEM((B,), jnp.float32),
        pltpu.VMEM((B, D), jnp.float32),
    ]
    slot_sems = [pltpu.SemaphoreType.DMA] * 5

    @functools.partial(
        pl.kernel,
        out_type=jax.ShapeDtypeStruct((NCORES, N, D), jnp.float32),
        mesh=mesh,
        compiler_params=pltpu.CompilerParams(needs_layout_passes=False),
        scratch_types=(slot_vmem * 4
                       + [pltpu.VMEM_SHARED((N, D), jnp.float32)]
                       + slot_sems * 4),
    )
    def sck(row_hbm, col_hbm, ex_hbm, xv_hbm, z_hbm, acc_out,
            r0, c0, w0, e0, v0, r1, c1, w1, e1, v1,
            r2, c2, w2, e2, v2, r3, c3, w3, e3, v3, acc,
            ir0, ic0, ie0, g0, s0, ir1, ic1, ie1, g1, s1,
            ir2, ic2, ie2, g2, s2, ir3, ic3, ie3, g3, s3):
        cid = lax.axis_index("c")
        sid = lax.axis_index("s")
        wid = cid * NSUB + sid

        zrows = 624
        zoff = sid * zrows
        pltpu.sync_copy(z_hbm.at[pl.ds(zoff, zrows)],
                        acc.at[pl.ds(zoff, zrows)])

        @pl.when(sid == NSUB - 1)
        def _():
            pltpu.sync_copy(z_hbm.at[pl.ds(zrows * NSUB, N - zrows * NSUB)],
                            acc.at[pl.ds(zrows * NSUB, N - zrows * NSUB)])

        plsc.subcore_barrier()

        base = wid * EPT
        bufs = (
            (r0, c0, w0, e0, v0, ir0, ic0, ie0, g0, s0),
            (r1, c1, w1, e1, v1, ir1, ic1, ie1, g1, s1),
            (r2, c2, w2, e2, v2, ir2, ic2, ie2, g2, s2),
            (r3, c3, w3, e3, v3, ir3, ic3, ie3, g3, s3),
        )

        def issue_idx(s, ci):
            idxr, idxc, _, exb, _, ir, ic, ie, _, _ = bufs[s]
            off = base + ci * B
            pltpu.async_copy(row_hbm.at[pl.ds(off, B)], idxr, ir)
            pltpu.async_copy(col_hbm.at[pl.ds(off, B)], idxc, ic)
            pltpu.async_copy(ex_hbm.at[pl.ds(off, B)], exb, ie)

        def wait_idx(s):
            idxr, idxc, _, exb, _, ir, ic, ie, _, _ = bufs[s]
            pltpu.make_async_copy(row_hbm.at[pl.ds(0, B)], idxr, ir).wait()
            pltpu.make_async_copy(col_hbm.at[pl.ds(0, B)], idxc, ic).wait()
            pltpu.make_async_copy(ex_hbm.at[pl.ds(0, B)], exb, ie).wait()

        def issue_gather(s):
            idxc, rows, g = bufs[s][1], bufs[s][4], bufs[s][8]
            pltpu.async_copy(xv_hbm.at[idxc], rows, g)

        def wait_gather(s):
            idxc, rows, g = bufs[s][1], bufs[s][4], bufs[s][8]
            pltpu.make_async_copy(xv_hbm.at[idxc], rows, g).wait()

        def issue_scatter(s):
            srow, rows, sem = bufs[s][2], bufs[s][4], bufs[s][9]
            pltpu.async_copy(rows, acc.at[srow], sem, add=True)

        def wait_scatter(s):
            srow, rows, sem = bufs[s][2], bufs[s][4], bufs[s][9]
            pltpu.make_async_copy(rows, acc.at[srow], sem).wait()

        def stash(s):
            idxr, srow = bufs[s][0], bufs[s][2]
            for g in range(NG):
                srow[pl.ds(g * 16, 16)] = idxr[pl.ds(g * 16, 16)]

        def scale(s):
            exb, rows = bufs[s][3], bufs[s][4]

            def body(g, c2):
                ex16 = exb[pl.ds(g * 16, 16)]
                for l in range(16):
                    b = g * 16 + l
                    exs = ex16[l]
                    for j in range(D // 16):
                        rows[b, pl.ds(j * 16, 16)] = (
                            rows[b, pl.ds(j * 16, 16)] * exs
                        )
                return c2

            lax.fori_loop(0, NG, body, 0)

        def step(i, k, first, last):
            wait_gather(k)
            stash(k)
            scale(k)
            kp2 = (k + 2) % 4
            if first:
                @pl.when(i >= 2)
                def _():
                    wait_scatter(kp2)
            else:
                wait_scatter(kp2)
            issue_scatter(k)

            if last:
                @pl.when(i + 2 < NCH)
                def _():
                    wait_idx(kp2)
                    issue_gather(kp2)
            else:
                wait_idx(kp2)
                issue_gather(kp2)

            @pl.when(i + 4 < NCH)
            def _():
                issue_idx(k, i + 4)

        for k in range(4):
            issue_idx(k, k)
        wait_idx(0)
        issue_gather(0)
        wait_idx(1)
        issue_gather(1)

        def quad(t, carry):
            for k in range(4):
                i = 4 * t + k
                step(i, k, first=(k < 2), last=(k >= 2))
            return carry

        lax.fori_loop(0, NCH // 4, quad, 0, unroll=False)
        i_last = NCH - 1
        wait_gather(0)
        stash(0)
        scale(0)
        wait_scatter(2)
        issue_scatter(0)
        wait_scatter(3)
        wait_scatter(0)
        plsc.subcore_barrier()

        @pl.when(sid == 0)
        def _():
            pltpu.sync_copy(acc, acc_out.at[cid])

    return sck(row, col, ex, xv, zeros_init)


def _tc_back(partials, s_part, ones32, bias):
    def body(p_ref, sp_ref, o32_ref, b_ref, o_ref):
        num = p_ref[0] + p_ref[1]
        s = lax.dot_general(
            sp_ref[...], o32_ref[...], (((0,), (0,)), ((), ())),
            preferred_element_type=jnp.float32,
        )
        r = jnp.where(s > 0.0, 1.0 / s, 0.0)
        o_ref[...] = num * r + b_ref[...]

    return pl.pallas_call(
        body,
        out_shape=jax.ShapeDtypeStruct((N, D), jnp.float32),
    )(partials, s_part, ones32, bias)


def kernel(x, edge_index, W_v, a_q, a_k, bias):
    row = jnp.asarray(edge_index[:, 0], dtype=jnp.int32)
    col = jnp.asarray(edge_index[:, 1], dtype=jnp.int32)
    xv, q2, k2 = _tc_front(x, W_v, a_q.reshape(D, 1), a_k.reshape(D, 1))
    q = q2.reshape(N)
    k = k2.reshape(N)
    ex, s_part = _sc_logits(row, col, q, k)
    zeros_init = jnp.zeros((N, D), dtype=jnp.float32)
    partials = _sc_spmm(row, col, ex, xv, zeros_init)
    ones32 = jnp.ones((NTILES, 1), dtype=jnp.float32)
    return _tc_back(partials, s_part, ones32, bias)
